# baseline (speedup 1.0000x reference)
import functools

import jax
import jax.numpy as jnp
from jax import lax
from jax.experimental import pallas as pl
from jax.experimental.pallas import tpu as pltpu
from jax.experimental.pallas import tpu_sc as plsc

D_MODEL = 64
N_TOK = 1000000
B_TOTAL = 4096 * 50

_info = plsc.get_sparse_core_info()
_NC, _NS = _info.num_cores, _info.num_subcores
_NW = _NC * _NS

_mesh = plsc.VectorSubcoreMesh(core_axis_name="c", subcore_axis_name="s")


_TBLK = 128
_NFULL = N_TOK // _TBLK
_TAIL = N_TOK - _NFULL * _TBLK
_BLK_F32 = _TBLK * D_MODEL
_N_PAIRS = (_NFULL // _NW + 2) // 2 + 1


@functools.partial(
    pl.kernel,
    out_type=jax.ShapeDtypeStruct((N_TOK * D_MODEL,), jnp.float32),
    mesh=_mesh,
    scratch_types=[
        pltpu.VMEM((2, D_MODEL, _TBLK), jnp.float32),
        pltpu.VMEM((D_MODEL, _TAIL), jnp.float32),
        pltpu.VMEM((_BLK_F32,), jnp.float32),
        pltpu.VMEM((_BLK_F32,), jnp.float32),
        pltpu.SemaphoreType.DMA,
        pltpu.SemaphoreType.DMA,
        pltpu.SemaphoreType.DMA,
        pltpu.SemaphoreType.DMA,
    ],
    compiler_params=pltpu.CompilerParams(
        use_tc_tiling_on_sc=True, needs_layout_passes=False),
)
def _compact_kernel(wt_hbm, out_hbm, win_v, tailwin_v, obuf0_v, obuf1_v,
                    isem0, isem1, osem0, osem1):
    wid = lax.axis_index("s") * _NC + lax.axis_index("c")
    obufs = (obuf0_v, obuf1_v)
    isems = (isem0, isem1)
    osems = (osem0, osem1)

    iota = lax.iota(jnp.int32, 16)
    dvecs = [d0 + iota for d0 in range(0, D_MODEL, 16)]

    def blk_of(i):
        return lax.rem(wid + _NW * i, _NFULL)

    def in_copy(i, b):
        return pltpu.make_async_copy(
            wt_hbm.at[:, pl.ds(blk_of(i) * _TBLK, _TBLK)],
            win_v.at[b], isems[b])

    def out_copy(i, b):
        return pltpu.make_async_copy(
            obufs[b], out_hbm.at[pl.ds(blk_of(i) * _BLK_F32, _BLK_F32)],
            osems[b])

    def transpose_into(src_ref, dst_ref, n_tok):
        for tt in range(n_tok):
            tvec = jnp.full((16,), tt, jnp.int32)
            for k, dvec in enumerate(dvecs):
                vals = plsc.load_gather(src_ref, [dvec, tvec])
                dst_ref[pl.ds(tt * D_MODEL + k * 16, 16)] = vals

    in_copy(0, 0).start()
    in_copy(1, 1).start()

    def step(i, b, first):
        in_copy(i, b).wait()
        if not first:
            out_copy(i - 2, b).wait()
        transpose_into(win_v.at[b], obufs[b], _TBLK)
        out_copy(i, b).start()
        in_copy(i + 2, b).start()

    step(0, 0, True)
    step(1, 1, True)

    def body(h, carry):
        step(2 * h, 0, False)
        step(2 * h + 1, 1, False)
        return carry

    lax.fori_loop(1, _N_PAIRS, body, 0)

    last = 2 * _N_PAIRS - 1
    in_copy(last + 1, 0).wait()
    in_copy(last + 2, 1).wait()
    out_copy(last - 1, 0).wait()
    out_copy(last, 1).wait()

    @pl.when(wid == _NW - 1)
    def _():
        tail0 = _NFULL * _TBLK
        pltpu.async_copy(
            wt_hbm.at[:, pl.ds(tail0, _TAIL)], tailwin_v, isem0).wait()
        transpose_into(tailwin_v, obuf0_v, _TAIL)
        pltpu.async_copy(
            obuf0_v.at[pl.ds(0, _TAIL * D_MODEL)],
            out_hbm.at[pl.ds(tail0 * D_MODEL, _TAIL * D_MODEL)],
            osem0).wait()


_B_PER_W = B_TOTAL // _NW
_CHUNK = 128
_N_CHUNKS = _B_PER_W // _CHUNK
_GROUP = 5
_G_ROWS = _GROUP * _CHUNK
_N_GROUPS = _N_CHUNKS // _GROUP


@functools.partial(
    pl.kernel,
    out_type=jax.ShapeDtypeStruct((B_TOTAL, D_MODEL), jnp.float32),
    mesh=_mesh,
    scratch_types=[
        pltpu.VMEM((_B_PER_W,), jnp.int32),
        pltpu.VMEM((2, _G_ROWS, D_MODEL), jnp.float32),
        pltpu.SemaphoreType.DMA,
        pltpu.SemaphoreType.DMA,
        pltpu.SemaphoreType.DMA,
    ],
    compiler_params=pltpu.CompilerParams(use_tc_tiling_on_sc=False),
)
def _gather_kernel(src_hbm, table_hbm, out_hbm, idx_v, rows_v, gsem,
                   osem0, osem1):
    wid = lax.axis_index("s") * _NC + lax.axis_index("c")
    row_base = wid * _B_PER_W

    pltpu.async_copy(src_hbm.at[pl.ds(row_base, _B_PER_W)], idx_v, gsem).wait()

    osems = (osem0, osem1)

    def out_copy(g, b):
        return pltpu.make_async_copy(
            rows_v.at[b],
            out_hbm.at[pl.ds(row_base + g * _G_ROWS, _G_ROWS)],
            osems[b])

    def fill(g, b):
        copies = [
            pltpu.async_copy(
                table_hbm.at[idx_v.at[pl.ds((g * _GROUP + j) * _CHUNK, _CHUNK)]],
                rows_v.at[b, pl.ds(j * _CHUNK, _CHUNK)],
                gsem)
            for j in range(_GROUP)
        ]
        for c in copies:
            c.wait()
        out_copy(g, b).start()

    fill(0, 0)
    fill(1, 1)

    def body(h, carry):
        g = 2 * h
        out_copy(g - 2, 0).wait()
        fill(g, 0)
        out_copy(g - 1, 1).wait()
        fill(g + 1, 1)
        return carry

    lax.fori_loop(1, _N_GROUPS // 2, body, 0)
    out_copy(_N_GROUPS - 2, 0).wait()
    out_copy(_N_GROUPS - 1, 1).wait()


def kernel(src, weight):
    wt = weight.T
    compact = _compact_kernel(wt)
    table = compact.reshape(N_TOK, D_MODEL)
    flat = src.reshape(-1).astype(jnp.int32)
    out = _gather_kernel(flat, table)
    return out.reshape(src.shape[0], src.shape[1], D_MODEL)


# speedup vs baseline: 1.2581x; 1.2581x over previous
import functools

import jax
import jax.numpy as jnp
from jax import lax
from jax.experimental import pallas as pl
from jax.experimental.pallas import tpu as pltpu
from jax.experimental.pallas import tpu_sc as plsc

D_MODEL = 64
N_TOK = 1000000
B_TOTAL = 4096 * 50

_info = plsc.get_sparse_core_info()
_NC, _NS = _info.num_cores, _info.num_subcores
_NW = _NC * _NS

_mesh = plsc.VectorSubcoreMesh(core_axis_name="c", subcore_axis_name="s")


_TBLK = 128
_NFULL = N_TOK // _TBLK
_TAIL = N_TOK - _NFULL * _TBLK
_BLK_F32 = _TBLK * D_MODEL
_N_PAIRS = (_NFULL // _NW + 2) // 2 + 1


@functools.partial(
    pl.kernel,
    out_type=jax.ShapeDtypeStruct((N_TOK * D_MODEL,), jnp.float32),
    mesh=_mesh,
    scratch_types=[
        pltpu.VMEM((2, D_MODEL, _TBLK), jnp.float32),
        pltpu.VMEM((D_MODEL, _TAIL), jnp.float32),
        pltpu.VMEM((_BLK_F32,), jnp.float32),
        pltpu.VMEM((_BLK_F32,), jnp.float32),
        pltpu.SemaphoreType.DMA,
        pltpu.SemaphoreType.DMA,
        pltpu.SemaphoreType.DMA,
        pltpu.SemaphoreType.DMA,
    ],
    compiler_params=pltpu.CompilerParams(
        use_tc_tiling_on_sc=True, needs_layout_passes=False),
)
def _compact_kernel(wt_hbm, out_hbm, win_v, tailwin_v, obuf0_v, obuf1_v,
                    isem0, isem1, osem0, osem1):
    wid = lax.axis_index("s") * _NC + lax.axis_index("c")
    obufs = (obuf0_v, obuf1_v)
    isems = (isem0, isem1)
    osems = (osem0, osem1)

    iota = lax.iota(jnp.int32, 16)
    qvecs = [(tt0 + iota) * D_MODEL for tt0 in range(0, _TBLK, 16)]

    def blk_of(i):
        return lax.rem(wid + _NW * i, _NFULL)

    def in_copy(i, b):
        return pltpu.make_async_copy(
            wt_hbm.at[:, pl.ds(blk_of(i) * _TBLK, _TBLK)],
            win_v.at[b], isems[b])

    def out_copy(i, b):
        return pltpu.make_async_copy(
            obufs[b], out_hbm.at[pl.ds(blk_of(i) * _BLK_F32, _BLK_F32)],
            osems[b])

    def transpose_into(src_ref, dst_ref, n_tok):
        for d in range(D_MODEL):
            for k in range(n_tok // 16):
                vals = src_ref[d, pl.ds(k * 16, 16)]
                plsc.store_scatter(dst_ref, [qvecs[k] + d], vals)

    in_copy(0, 0).start()
    in_copy(1, 1).start()

    def step(i, b, first):
        in_copy(i, b).wait()
        if not first:
            out_copy(i - 2, b).wait()
        transpose_into(win_v.at[b], obufs[b], _TBLK)
        out_copy(i, b).start()
        in_copy(i + 2, b).start()

    step(0, 0, True)
    step(1, 1, True)

    def body(h, carry):
        step(2 * h, 0, False)
        step(2 * h + 1, 1, False)
        return carry

    lax.fori_loop(1, _N_PAIRS, body, 0)

    last = 2 * _N_PAIRS - 1
    in_copy(last + 1, 0).wait()
    in_copy(last + 2, 1).wait()
    out_copy(last - 1, 0).wait()
    out_copy(last, 1).wait()

    @pl.when(wid == _NW - 1)
    def _():
        tail0 = _NFULL * _TBLK
        pltpu.async_copy(
            wt_hbm.at[:, pl.ds(tail0, _TAIL)], tailwin_v, isem0).wait()
        transpose_into(tailwin_v, obuf0_v, _TAIL)
        pltpu.async_copy(
            obuf0_v.at[pl.ds(0, _TAIL * D_MODEL)],
            out_hbm.at[pl.ds(tail0 * D_MODEL, _TAIL * D_MODEL)],
            osem0).wait()


_B_PER_W = B_TOTAL // _NW
_CHUNK = 128
_N_CHUNKS = _B_PER_W // _CHUNK
_GROUP = 5
_G_ROWS = _GROUP * _CHUNK
_N_GROUPS = _N_CHUNKS // _GROUP


@functools.partial(
    pl.kernel,
    out_type=jax.ShapeDtypeStruct((B_TOTAL, D_MODEL), jnp.float32),
    mesh=_mesh,
    scratch_types=[
        pltpu.VMEM((_B_PER_W,), jnp.int32),
        pltpu.VMEM((2, _G_ROWS, D_MODEL), jnp.float32),
        pltpu.SemaphoreType.DMA,
        pltpu.SemaphoreType.DMA,
        pltpu.SemaphoreType.DMA,
    ],
    compiler_params=pltpu.CompilerParams(use_tc_tiling_on_sc=False),
)
def _gather_kernel(src_hbm, table_hbm, out_hbm, idx_v, rows_v, gsem,
                   osem0, osem1):
    wid = lax.axis_index("s") * _NC + lax.axis_index("c")
    row_base = wid * _B_PER_W

    pltpu.async_copy(src_hbm.at[pl.ds(row_base, _B_PER_W)], idx_v, gsem).wait()

    osems = (osem0, osem1)

    def out_copy(g, b):
        return pltpu.make_async_copy(
            rows_v.at[b],
            out_hbm.at[pl.ds(row_base + g * _G_ROWS, _G_ROWS)],
            osems[b])

    def fill(g, b):
        copies = [
            pltpu.async_copy(
                table_hbm.at[idx_v.at[pl.ds((g * _GROUP + j) * _CHUNK, _CHUNK)]],
                rows_v.at[b, pl.ds(j * _CHUNK, _CHUNK)],
                gsem)
            for j in range(_GROUP)
        ]
        for c in copies:
            c.wait()
        out_copy(g, b).start()

    fill(0, 0)
    fill(1, 1)

    def body(h, carry):
        g = 2 * h
        out_copy(g - 2, 0).wait()
        fill(g, 0)
        out_copy(g - 1, 1).wait()
        fill(g + 1, 1)
        return carry

    lax.fori_loop(1, _N_GROUPS // 2, body, 0)
    out_copy(_N_GROUPS - 2, 0).wait()
    out_copy(_N_GROUPS - 1, 1).wait()


def kernel(src, weight):
    wt = weight.T
    compact = _compact_kernel(wt)
    table = compact.reshape(N_TOK, D_MODEL)
    flat = src.reshape(-1).astype(jnp.int32)
    out = _gather_kernel(flat, table)
    return out.reshape(src.shape[0], src.shape[1], D_MODEL)


# speedup vs baseline: 2.3138x; 1.8391x over previous
import functools

import jax
import jax.numpy as jnp
from jax import lax
from jax.experimental import pallas as pl
from jax.experimental.pallas import tpu as pltpu
from jax.experimental.pallas import tpu_sc as plsc

D_MODEL = 64
B_TOTAL = 4096 * 50

_info = plsc.get_sparse_core_info()
_NC, _NS = _info.num_cores, _info.num_subcores
_NW = _NC * _NS
_B_PER_W = B_TOTAL // _NW
_CHUNK = 128
_N_CHUNKS = _B_PER_W // _CHUNK
_GROUP = 5
_G_ROWS = _GROUP * _CHUNK
_N_GROUPS = _N_CHUNKS // _GROUP

_mesh = plsc.VectorSubcoreMesh(core_axis_name="c", subcore_axis_name="s")


@functools.partial(
    pl.kernel,
    out_type=jax.ShapeDtypeStruct((B_TOTAL, D_MODEL), jnp.float32),
    mesh=_mesh,
    scratch_types=[
        pltpu.VMEM((_B_PER_W,), jnp.int32),
        pltpu.VMEM((2, _G_ROWS, D_MODEL), jnp.float32),
        pltpu.SemaphoreType.DMA,
        pltpu.SemaphoreType.DMA,
        pltpu.SemaphoreType.DMA,
    ],
    compiler_params=pltpu.CompilerParams(use_tc_tiling_on_sc=False),
)
def _gather_kernel(src_hbm, table_hbm, out_hbm, idx_v, rows_v, gsem,
                   osem0, osem1):
    wid = lax.axis_index("s") * _NC + lax.axis_index("c")
    row_base = wid * _B_PER_W

    pltpu.async_copy(src_hbm.at[pl.ds(row_base, _B_PER_W)], idx_v, gsem).wait()

    osems = (osem0, osem1)

    def out_copy(g, b):
        return pltpu.make_async_copy(
            rows_v.at[b],
            out_hbm.at[pl.ds(row_base + g * _G_ROWS, _G_ROWS)],
            osems[b])

    def fill(g, b):
        copies = [
            pltpu.async_copy(
                table_hbm.at[idx_v.at[pl.ds((g * _GROUP + j) * _CHUNK, _CHUNK)]],
                rows_v.at[b, pl.ds(j * _CHUNK, _CHUNK)],
                gsem)
            for j in range(_GROUP)
        ]
        for c in copies:
            c.wait()
        out_copy(g, b).start()

    fill(0, 0)
    fill(1, 1)

    def body(h, carry):
        g = 2 * h
        out_copy(g - 2, 0).wait()
        fill(g, 0)
        out_copy(g - 1, 1).wait()
        fill(g + 1, 1)
        return carry

    lax.fori_loop(1, _N_GROUPS // 2, body, 0)
    out_copy(_N_GROUPS - 2, 0).wait()
    out_copy(_N_GROUPS - 1, 1).wait()


def kernel(src, weight):
    flat = src.reshape(-1).astype(jnp.int32)
    out = _gather_kernel(flat, weight)
    return out.reshape(src.shape[0], src.shape[1], D_MODEL)
